# baseline (speedup 1.0000x reference)
import jax
import jax.numpy as jnp
from jax import lax
from jax.experimental import pallas as pl
from jax.experimental.pallas import tpu as pltpu
from jax.experimental.pallas import tpu_sc as plsc

BATCH = 16384
EMBED_DIM = 32
NUM_CORES = 2
NUM_SUBCORES = 16
NUM_WORKERS = NUM_CORES * NUM_SUBCORES
B_PER_W = BATCH // NUM_WORKERS
LANES = 16
GROUPS = B_PER_W // LANES


def _body(user_ids_hbm, item_ids_hbm, user_table_hbm, item_table_hbm,
          out_hbm, uid_v, iid_v, urows_v, irows_v, prod_v, out_v, sem):
    wid = lax.axis_index("s") * NUM_CORES + lax.axis_index("c")
    base = wid * B_PER_W

    pltpu.sync_copy(user_ids_hbm.at[pl.ds(base, B_PER_W)], uid_v)
    pltpu.sync_copy(item_ids_hbm.at[pl.ds(base, B_PER_W)], iid_v)

    cp_u = pltpu.async_copy(user_table_hbm.at[uid_v], urows_v, sem)
    cp_i = pltpu.async_copy(item_table_hbm.at[iid_v], irows_v, sem)
    cp_u.wait()
    cp_i.wait()

    lane = lax.iota(jnp.int32, LANES)
    lane_stride = lane * EMBED_DIM

    def group(g, carry):
        row0 = g * LANES
        for b in range(LANES):
            for h in range(EMBED_DIM // LANES):
                u = urows_v[row0 + b, pl.ds(h * LANES, LANES)]
                v = irows_v[row0 + b, pl.ds(h * LANES, LANES)]
                prod_v[pl.ds(b * EMBED_DIM + h * LANES, LANES)] = u * v
        acc = jnp.zeros((LANES,), jnp.float32)
        for d in range(EMBED_DIM):
            acc = acc + plsc.load_gather(prod_v, [lane_stride + d])
        out_v[pl.ds(row0, LANES)] = acc
        return carry

    lax.fori_loop(0, GROUPS, group, 0)

    pltpu.sync_copy(out_v, out_hbm.at[pl.ds(base, B_PER_W)])


@jax.jit
def kernel(user_ids, item_ids, user_table, item_table):
    mesh = plsc.VectorSubcoreMesh(core_axis_name="c", subcore_axis_name="s")
    f = pl.kernel(
        _body,
        mesh=mesh,
        compiler_params=pltpu.CompilerParams(
            needs_layout_passes=False, use_tc_tiling_on_sc=False),
        out_type=jax.ShapeDtypeStruct((BATCH,), jnp.float32),
        scratch_types=[
            pltpu.VMEM((B_PER_W,), jnp.int32),
            pltpu.VMEM((B_PER_W,), jnp.int32),
            pltpu.VMEM((B_PER_W, EMBED_DIM), jnp.float32),
            pltpu.VMEM((B_PER_W, EMBED_DIM), jnp.float32),
            pltpu.VMEM((LANES * EMBED_DIM,), jnp.float32),
            pltpu.VMEM((B_PER_W,), jnp.float32),
            pltpu.SemaphoreType.DMA,
        ],
    )
    return f(user_ids.astype(jnp.int32), item_ids.astype(jnp.int32),
             user_table, item_table)
